# baseline (speedup 1.0000x reference)
import functools

import jax
import jax.numpy as jnp
from jax import lax
from jax.experimental import pallas as pl
from jax.experimental.pallas import tpu as pltpu
from jax.experimental.pallas import tpu_sc as plsc

B = 4096
S = 200
D = 64
H = 128
O = 16

L = 16
NC = 2
NS = 16
NW = NC * NS
ROWS_W = B // NW
HALF = S // 2
CHUNKS_W = 2 * ROWS_W
D_CH = D // L


def _sc_pool(idx, table):
    mesh = plsc.VectorSubcoreMesh(core_axis_name="c", subcore_axis_name="s")

    @functools.partial(
        pl.kernel,
        out_type=jax.ShapeDtypeStruct((B, D), jnp.float32),
        mesh=mesh,
        scratch_types=[
            pltpu.VMEM((CHUNKS_W, HALF), jnp.int32),
            pltpu.VMEM((2, S, D), jnp.float32),
            pltpu.VMEM((ROWS_W, D), jnp.float32),
            pltpu.SemaphoreType.DMA,
            pltpu.SemaphoreType.DMA,
        ],
        compiler_params=pltpu.CompilerParams(use_tc_tiling_on_sc=False),
    )
    def pool(idx_hbm, table_hbm, out_hbm, idx_v, rows_v, pooled_v, sem0, sem1):
        sems = (sem0, sem1)
        wid = lax.axis_index("s") * NC + lax.axis_index("c")
        pltpu.sync_copy(idx_hbm.at[wid], idx_v)

        def fire(row, p):
            for j in range(2):
                pltpu.async_copy(
                    table_hbm.at[idx_v.at[2 * row + j]],
                    rows_v.at[p, pl.ds(j * HALF, HALF)],
                    sems[p])

        def drain(row, p):
            for j in range(2):
                pltpu.make_async_copy(
                    table_hbm.at[idx_v.at[2 * row + j]],
                    rows_v.at[p, pl.ds(j * HALF, HALF)],
                    sems[p]).wait()

        fire(0, 0)

        @pl.loop(0, ROWS_W, step=2)
        def _(i):
            for p in range(2):
                row = i + p

                @pl.when(row + 1 < ROWS_W)
                def _():
                    fire(row + 1, 1 - p)

                drain(row, p)

                def body(r, acc):
                    return tuple(
                        acc[c] + rows_v[p, r, pl.ds(c * L, L)]
                        for c in range(D_CH))

                acc = lax.fori_loop(
                    0, S, body,
                    tuple(jnp.zeros((L,), jnp.float32) for _ in range(D_CH)))
                for c in range(D_CH):
                    pooled_v[row, pl.ds(c * L, L)] = acc[c] * (1.0 / S)

        pltpu.sync_copy(pooled_v, out_hbm.at[pl.ds(wid * ROWS_W, ROWS_W)])

    return pool(idx, table)


BB = 512


def _mlp(x, W1, b1, W2, b2):
    def body(x_ref, w1_ref, b1_ref, w2_ref, b2_ref, o_ref):
        h = jnp.dot(x_ref[...], w1_ref[...],
                    preferred_element_type=jnp.float32) + b1_ref[...]
        z = jnp.dot(h, w2_ref[...],
                    preferred_element_type=jnp.float32) + b2_ref[...]
        z = z - jnp.max(z, axis=-1, keepdims=True)
        e = jnp.exp(z)
        o_ref[...] = e / jnp.sum(e, axis=-1, keepdims=True)

    return pl.pallas_call(
        body,
        grid=(B // BB,),
        in_specs=[
            pl.BlockSpec((BB, D), lambda i: (i, 0)),
            pl.BlockSpec((D, H), lambda i: (0, 0)),
            pl.BlockSpec((1, H), lambda i: (0, 0)),
            pl.BlockSpec((H, O), lambda i: (0, 0)),
            pl.BlockSpec((1, O), lambda i: (0, 0)),
        ],
        out_specs=pl.BlockSpec((BB, O), lambda i: (i, 0)),
        out_shape=jax.ShapeDtypeStruct((B, O), jnp.float32),
    )(x, W1, b1.reshape(1, H), W2, b2.reshape(1, O))


def kernel(inputs, table, W1, b1, W2, b2):
    idx = inputs.astype(jnp.int32).reshape(NW, CHUNKS_W, HALF)
    pooled = _sc_pool(idx, table)
    return _mlp(pooled, W1, b1, W2, b2)
